# Initial kernel scaffold; baseline (speedup 1.0000x reference)
import jax
import jax.numpy as jnp
from jax.experimental import pallas as pl


def kernel(x, edge_index, W1, b1, W2, b2, W3, b3):
    raise NotImplementedError("write your pallas kernel here")


# speedup vs baseline: 15.8849x; 15.8849x over previous
import functools

import jax
import jax.numpy as jnp
from jax import lax
from jax.experimental import pallas as pl
from jax.experimental.pallas import tpu as pltpu
from jax.experimental.pallas import tpu_sc as plsc

NC = 2
NS = 16
NW = NC * NS
L = 16

K = 80
ZR = 128

_MESH = plsc.VectorSubcoreMesh(core_axis_name="c", subcore_axis_name="s")


def _deg_kernel_body(npad, ch_w, cols_hbm, out_hbm, cidx, hist, accs, tmps,
                     stage):
    c = lax.axis_index("c")
    s = lax.axis_index("s")
    w = c * NS + s
    zeros16 = jnp.zeros((L,), jnp.float32)
    ones16 = jnp.ones((L,), jnp.float32)

    def zb(i, carry):
        hist[pl.ds(i * L, L)] = zeros16
        return carry

    lax.fori_loop(0, 3 * npad // L, zb, None)

    for k in range(3):
        def ebody(j, carry, k=k):
            cid = w * ch_w + j
            pltpu.sync_copy(cols_hbm.at[k, cid], cidx)
            for i in range(K // L):
                idx16 = cidx[pl.ds(i * L, L)] + (k * npad)
                plsc.addupdate_scatter(hist, [idx16], ones16)
            return carry

        lax.fori_loop(0, ch_w, ebody, None)

    pltpu.sync_copy(hist, stage.at[s])
    plsc.subcore_barrier()

    sl = 3 * npad // NS

    def zb2(i, carry):
        accs[pl.ds(i * L, L)] = zeros16
        return carry

    lax.fori_loop(0, sl // L, zb2, None)
    for j2 in range(NS):
        pltpu.sync_copy(stage.at[j2, pl.ds(s * sl, sl)], tmps)

        def ab(i, carry):
            accs[pl.ds(i * L, L)] = accs[pl.ds(i * L, L)] + tmps[pl.ds(i * L, L)]
            return carry

        lax.fori_loop(0, sl // L, ab, None)
    pltpu.sync_copy(accs, out_hbm.at[c, pl.ds(s * sl, sl)])


def _scatter_kernel_body(np_r, d, ch, ch_w, rows_hbm, cols_hbm, g1, g2, g3,
                         out_hbm, ridx, cidx, rbuf, zbuf, sem, acc):
    c = lax.axis_index("c")
    s = lax.axis_index("s")
    zeros16 = jnp.zeros((L,), jnp.float32)

    def zrow(r, carry):
        for colj in range(d // L):
            zbuf[r, pl.ds(colj * L, L)] = zeros16
        return carry

    lax.fori_loop(0, ZR, zrow, None)

    rows_t = np_r // NS
    nz = rows_t // ZR
    glist = [g1, g2, g3]
    for k in range(3):
        for z in range(nz):
            pltpu.sync_copy(zbuf, acc.at[pl.ds(s * rows_t + z * ZR, ZR)])
        plsc.subcore_barrier()

        def ebody(j, carry, k=k, gk=glist[k]):
            cid = c * (ch // NC) + s * ch_w + j
            pltpu.sync_copy(rows_hbm.at[k, cid], ridx)
            pltpu.sync_copy(cols_hbm.at[k, cid], cidx)
            pltpu.async_copy(gk.at[ridx], rbuf, sem).wait()
            pltpu.sync_copy(rbuf, acc.at[cidx], add=True)
            return carry

        lax.fori_loop(0, ch_w, ebody, None)
        plsc.subcore_barrier()
        for z in range(nz):
            rsl = pl.ds(s * rows_t + z * ZR, ZR)
            pltpu.sync_copy(acc.at[rsl], out_hbm.at[k, c, rsl])
        plsc.subcore_barrier()


def _mm_body(x_ref, w_ref, dp_ref, g1_ref, g2_ref, g3_ref):
    xb = x_ref[...]
    grefs = [g1_ref, g2_ref, g3_ref]
    for k in range(3):
        deg = dp_ref[:, 0, k] + dp_ref[:, 1, k] + 1.0
        dis = lax.rsqrt(deg)[:, None]
        h = jnp.dot(xb, w_ref[k], preferred_element_type=jnp.float32)
        grefs[k][...] = h * dis


def _comb_body(p_ref, g1_ref, g2_ref, g3_ref, dp_ref, b_ref, o_ref):
    rb, d = o_ref.shape
    acc = jnp.broadcast_to(b_ref[0] + b_ref[1] + b_ref[2], (rb, d))
    grefs = [g1_ref, g2_ref, g3_ref]
    for k in range(3):
        dis = lax.rsqrt(dp_ref[:, 0, k] + dp_ref[:, 1, k] + 1.0)[:, None]
        acc = acc + dis * (p_ref[k, 0] + p_ref[k, 1] + grefs[k][...])
    o_ref[...] = acc


def kernel(x, edge_index, W1, b1, W2, b2, W3, b3):
    d0, d1, n, d = x.shape
    e = edge_index.shape[-1]
    npad = ((n + 255) // 256) * 256
    ch = e // K
    ch_w = ch // NW
    assert ch * K == e and ch_w * NW == ch
    assert npad % (NS * ZR) == 0 and d % L == 0

    x2d = x.reshape(n, d)
    ei32 = edge_index.astype(jnp.int32).reshape(3, 2, ch, K)
    rows_idx = ei32[:, 0]
    cols_idx = ei32[:, 1]
    wstack = jnp.stack([W1, W2, W3])
    bstack = jnp.stack([b1, b2, b3])

    deg_body = functools.partial(_deg_kernel_body, npad, ch_w)
    deg_part = pl.kernel(
        deg_body,
        out_type=jax.ShapeDtypeStruct((NC, 3 * npad), jnp.float32),
        mesh=_MESH,
        scratch_types=[
            pltpu.VMEM((K,), jnp.int32),
            pltpu.VMEM((3 * npad,), jnp.float32),
            pltpu.VMEM((3 * npad // NS,), jnp.float32),
            pltpu.VMEM((3 * npad // NS,), jnp.float32),
            pltpu.VMEM_SHARED((NS, 3 * npad), jnp.float32),
        ],
        compiler_params=pltpu.CompilerParams(needs_layout_passes=False),
        name="gcn_deg_sc",
    )(cols_idx)
    deg_t = jnp.transpose(deg_part.reshape(NC, 3, npad), (2, 0, 1))

    nblk = 10
    rb = n // nblk
    g1, g2, g3 = pl.pallas_call(
        _mm_body,
        grid=(nblk,),
        in_specs=[
            pl.BlockSpec((rb, d), lambda b: (b, 0)),
            pl.BlockSpec((3, d, d), lambda b: (0, 0, 0)),
            pl.BlockSpec((rb, NC, 3), lambda b: (b, 0, 0)),
        ],
        out_specs=[pl.BlockSpec((rb, d), lambda b: (b, 0))] * 3,
        out_shape=[jax.ShapeDtypeStruct((n, d), jnp.float32)] * 3,
        name="gcn_mm_tc",
    )(x2d, wstack, deg_t)

    sc_body = functools.partial(_scatter_kernel_body, npad, d, ch, ch_w)
    parts = pl.kernel(
        sc_body,
        out_type=jax.ShapeDtypeStruct((3, NC, npad, d), jnp.float32),
        mesh=_MESH,
        scratch_types=[
            pltpu.VMEM((K,), jnp.int32),
            pltpu.VMEM((K,), jnp.int32),
            pltpu.VMEM((K, d), jnp.float32),
            pltpu.VMEM((ZR, d), jnp.float32),
            pltpu.SemaphoreType.DMA,
            pltpu.VMEM_SHARED((npad, d), jnp.float32),
        ],
        compiler_params=pltpu.CompilerParams(needs_layout_passes=False),
        name="gcn_scatter_sc",
    )(rows_idx, cols_idx, g1, g2, g3)

    out2d = pl.pallas_call(
        _comb_body,
        grid=(nblk,),
        in_specs=[
            pl.BlockSpec((3, NC, rb, d), lambda b: (0, 0, b, 0)),
            pl.BlockSpec((rb, d), lambda b: (b, 0)),
            pl.BlockSpec((rb, d), lambda b: (b, 0)),
            pl.BlockSpec((rb, d), lambda b: (b, 0)),
            pl.BlockSpec((rb, NC, 3), lambda b: (b, 0, 0)),
            pl.BlockSpec((3, d), lambda b: (0, 0)),
        ],
        out_specs=pl.BlockSpec((rb, d), lambda b: (b, 0)),
        out_shape=jax.ShapeDtypeStruct((n, d), jnp.float32),
        name="gcn_combine_tc",
    )(parts, g1, g2, g3, deg_t, bstack)

    return out2d.reshape(d0, d1, n, d)


# speedup vs baseline: 33.8141x; 2.1287x over previous
import functools

import jax
import jax.numpy as jnp
from jax import lax
from jax.experimental import pallas as pl
from jax.experimental.pallas import tpu as pltpu
from jax.experimental.pallas import tpu_sc as plsc

NC = 2
NS = 16
NW = NC * NS
L = 16

K = 40
ZR = 128
NB = 5

_MESH = plsc.VectorSubcoreMesh(core_axis_name="c", subcore_axis_name="s")


def _deg_kernel_body(npad, ch_w, cols_hbm, out_hbm, cidx, hist, accs, tmps,
                     stage):
    c = lax.axis_index("c")
    s = lax.axis_index("s")
    w = c * NS + s
    zeros16 = jnp.zeros((L,), jnp.float32)
    ones16 = jnp.ones((L,), jnp.float32)

    def zb(i, carry):
        hist[pl.ds(i * L, L)] = zeros16
        return carry

    lax.fori_loop(0, 3 * npad // L, zb, None)

    ew = cols_hbm.shape[-1]
    for k in range(3):
        pltpu.sync_copy(cols_hbm.at[k, w, 0], cidx)

        def ebody(j, carry, k=k):
            idx16 = cidx[pl.ds(j * L, L)] + (k * npad)
            plsc.addupdate_scatter(hist, [idx16], ones16)
            return carry

        lax.fori_loop(0, ew // L, ebody, None)

    pltpu.sync_copy(hist, stage.at[s])
    plsc.subcore_barrier()

    sl = 3 * npad // NS

    def zb2(i, carry):
        accs[pl.ds(i * L, L)] = zeros16
        return carry

    lax.fori_loop(0, sl // L, zb2, None)
    for j2 in range(NS):
        pltpu.sync_copy(stage.at[j2, pl.ds(s * sl, sl)], tmps)

        def ab(i, carry):
            accs[pl.ds(i * L, L)] = accs[pl.ds(i * L, L)] + tmps[pl.ds(i * L, L)]
            return carry

        lax.fori_loop(0, sl // L, ab, None)
    pltpu.sync_copy(accs, out_hbm.at[c, pl.ds(s * sl, sl)])


def _scatter_kernel_body(np_r, d, nblk_o, rows_hbm, cols_hbm, zeros_hbm,
                         g1, g2, g3, out_hbm, ridx, cidx, rbuf, isem, gsem,
                         ssem, acc):
    c = lax.axis_index("c")
    s = lax.axis_index("s")
    w = c * NS + s
    rows_t = np_r // NS
    nz = rows_t // ZR
    glist = [g1, g2, g3]
    for k in range(3):
        pltpu.sync_copy(zeros_hbm, acc.at[pl.ds(s * rows_t, rows_t)])
        plsc.subcore_barrier()
        pltpu.async_copy(rows_hbm.at[k, w, 0], ridx.at[0], isem)
        pltpu.async_copy(cols_hbm.at[k, w, 0], cidx.at[0], isem)

        def outer(j0, carry, gk=glist[k]):
            cur = lax.rem(j0, 2)
            nxt = lax.rem(j0 + 1, 2)
            pltpu.make_async_copy(rows_hbm.at[k, w, 0], ridx.at[cur],
                                  isem).wait()
            pltpu.make_async_copy(cols_hbm.at[k, w, 0], cidx.at[cur],
                                  isem).wait()
            jn = jnp.minimum(j0 + 1, nblk_o - 1)
            pltpu.async_copy(rows_hbm.at[k, w, jn], ridx.at[nxt], isem)
            pltpu.async_copy(cols_hbm.at[k, w, jn], cidx.at[nxt], isem)
            gd = []
            for b in range(NB):
                gd.append(pltpu.async_copy(gk.at[ridx.at[cur, b]],
                                           rbuf.at[b], gsem))
            sd = []
            for b in range(NB):
                gd[b].wait()
                sd.append(pltpu.async_copy(rbuf.at[b],
                                           acc.at[cidx.at[cur, b]],
                                           ssem, add=True))
            for dsc in sd:
                dsc.wait()
            return carry

        lax.fori_loop(0, nblk_o, outer, None)
        pltpu.make_async_copy(rows_hbm.at[k, w, 0], ridx.at[0], isem).wait()
        pltpu.make_async_copy(cols_hbm.at[k, w, 0], cidx.at[0], isem).wait()
        plsc.subcore_barrier()
        for z in range(nz):
            rsl = pl.ds(s * rows_t + z * ZR, ZR)
            pltpu.sync_copy(acc.at[rsl], out_hbm.at[k, c, rsl])
        plsc.subcore_barrier()


def _mm_body(x_ref, w_ref, dp_ref, g1_ref, g2_ref, g3_ref):
    xb = x_ref[...]
    grefs = [g1_ref, g2_ref, g3_ref]
    for k in range(3):
        deg = dp_ref[:, 0, k] + dp_ref[:, 1, k] + 1.0
        dis = lax.rsqrt(deg)[:, None]
        h = jnp.dot(xb, w_ref[k], preferred_element_type=jnp.float32)
        grefs[k][...] = h * dis


def _comb_body(p_ref, g1_ref, g2_ref, g3_ref, dp_ref, b_ref, o_ref):
    rb, d = o_ref.shape
    acc = jnp.broadcast_to(b_ref[0] + b_ref[1] + b_ref[2], (rb, d))
    grefs = [g1_ref, g2_ref, g3_ref]
    for k in range(3):
        dis = lax.rsqrt(dp_ref[:, 0, k] + dp_ref[:, 1, k] + 1.0)[:, None]
        acc = acc + dis * (p_ref[k, 0] + p_ref[k, 1] + grefs[k][...])
    o_ref[...] = acc


def kernel(x, edge_index, W1, b1, W2, b2, W3, b3):
    d0, d1, n, d = x.shape
    e = edge_index.shape[-1]
    npad = ((n + 255) // 256) * 256
    ch = e // K
    ch_w = ch // NW
    assert ch * K == e and ch_w * NW == ch and ch_w % NB == 0
    assert npad % (NS * ZR) == 0 and d % L == 0
    nblk_o = ch_w // NB
    ew = ch_w * K

    x2d = x.reshape(n, d)
    ei32 = edge_index.astype(jnp.int32).reshape(3, 2, NW, nblk_o, NB, K)
    rows_idx = ei32[:, 0]
    cols_idx = ei32[:, 1]
    cols_flat = cols_idx.reshape(3, NW, 1, ew)
    zeros_rows = jnp.zeros((npad // NS, d), jnp.float32)
    wstack = jnp.stack([W1, W2, W3])
    bstack = jnp.stack([b1, b2, b3])

    deg_body = functools.partial(_deg_kernel_body, npad, ch_w)
    deg_part = pl.kernel(
        deg_body,
        out_type=jax.ShapeDtypeStruct((NC, 3 * npad), jnp.float32),
        mesh=_MESH,
        scratch_types=[
            pltpu.VMEM((ew,), jnp.int32),
            pltpu.VMEM((3 * npad,), jnp.float32),
            pltpu.VMEM((3 * npad // NS,), jnp.float32),
            pltpu.VMEM((3 * npad // NS,), jnp.float32),
            pltpu.VMEM_SHARED((NS, 3 * npad), jnp.float32),
        ],
        compiler_params=pltpu.CompilerParams(needs_layout_passes=False),
        name="gcn_deg_sc",
    )(cols_flat)
    deg_t = jnp.transpose(deg_part.reshape(NC, 3, npad), (2, 0, 1))

    nblk = 10
    rb = n // nblk
    g1, g2, g3 = pl.pallas_call(
        _mm_body,
        grid=(nblk,),
        in_specs=[
            pl.BlockSpec((rb, d), lambda b: (b, 0)),
            pl.BlockSpec((3, d, d), lambda b: (0, 0, 0)),
            pl.BlockSpec((rb, NC, 3), lambda b: (b, 0, 0)),
        ],
        out_specs=[pl.BlockSpec((rb, d), lambda b: (b, 0))] * 3,
        out_shape=[jax.ShapeDtypeStruct((n, d), jnp.float32)] * 3,
        name="gcn_mm_tc",
    )(x2d, wstack, deg_t)

    sc_body = functools.partial(_scatter_kernel_body, npad, d, nblk_o)
    parts = pl.kernel(
        sc_body,
        out_type=jax.ShapeDtypeStruct((3, NC, npad, d), jnp.float32),
        mesh=_MESH,
        scratch_types=[
            pltpu.VMEM((2, NB, K), jnp.int32),
            pltpu.VMEM((2, NB, K), jnp.int32),
            pltpu.VMEM((NB, K, d), jnp.float32),
            pltpu.SemaphoreType.DMA,
            pltpu.SemaphoreType.DMA,
            pltpu.SemaphoreType.DMA,
            pltpu.VMEM_SHARED((npad, d), jnp.float32),
        ],
        compiler_params=pltpu.CompilerParams(needs_layout_passes=False),
        name="gcn_scatter_sc",
    )(rows_idx, cols_idx, zeros_rows, g1, g2, g3)

    out2d = pl.pallas_call(
        _comb_body,
        grid=(nblk,),
        in_specs=[
            pl.BlockSpec((3, NC, rb, d), lambda b: (0, 0, b, 0)),
            pl.BlockSpec((rb, d), lambda b: (b, 0)),
            pl.BlockSpec((rb, d), lambda b: (b, 0)),
            pl.BlockSpec((rb, d), lambda b: (b, 0)),
            pl.BlockSpec((rb, NC, 3), lambda b: (b, 0, 0)),
            pl.BlockSpec((3, d), lambda b: (0, 0)),
        ],
        out_specs=pl.BlockSpec((rb, d), lambda b: (b, 0)),
        out_shape=jax.ShapeDtypeStruct((n, d), jnp.float32),
        name="gcn_combine_tc",
    )(parts, g1, g2, g3, deg_t, bstack)

    return out2d.reshape(d0, d1, n, d)
